# baseline (speedup 1.0000x reference)
import functools

import jax
import jax.numpy as jnp
from jax import lax
from jax.experimental import pallas as pl
from jax.experimental.pallas import tpu as pltpu
from jax.experimental.pallas import tpu_sc as plsc

T = 2048
D = 768
E = 16
K = 2
BLK = 128
NBLK = 48
N_PAD = NBLK * BLK
NW = 32
TPW = T // NW
APW = TPW * K
TCB = 512
D2 = D // 2


def _pack_bf16(a):
    ab = a.astype(jnp.bfloat16)
    lo = lax.bitcast_convert_type(ab[..., :D2], jnp.int16).astype(jnp.int32)
    hi = lax.bitcast_convert_type(ab[..., D2:], jnp.int16).astype(jnp.int32)
    return (hi << 16) | (lo & 0xFFFF)


def _unpack_bf16_raw(w):
    lo = lax.bitcast_convert_type(w.astype(jnp.int16), jnp.bfloat16)
    hi = lax.bitcast_convert_type((w >> 16).astype(jnp.int16), jnp.bfloat16)
    return jnp.concatenate([lo, hi], axis=-1)


def _unpack_bf16(w):
    return _unpack_bf16_raw(w).astype(jnp.float32)


def _gating_body(x_ref, wg_ref, p_ref, e_ref, hist_ref, xbf_ref,
                 emap_ref, runx_ref, rexp_ref, nrun_ref):
    x = x_ref[...]
    logits = jnp.dot(x, wg_ref[...], preferred_element_type=jnp.float32)
    m = jnp.max(logits, axis=-1, keepdims=True)
    ex = jnp.exp(logits - m)
    probs = ex / jnp.sum(ex, axis=-1, keepdims=True)

    iota = lax.broadcasted_iota(jnp.int32, (T, E), 1)
    p0 = jnp.max(probs, axis=-1)
    e0 = jnp.min(jnp.where(probs == p0[:, None], iota, E), axis=-1)
    probs2 = jnp.where(iota == e0[:, None], -jnp.inf, probs)
    p1 = jnp.max(probs2, axis=-1)
    e1 = jnp.min(jnp.where(probs2 == p1[:, None], iota, E), axis=-1)

    p_ref[...] = jnp.concatenate([p0[:, None], p1[:, None]], axis=1)
    e_ref[...] = jnp.concatenate([e0[None, :], e1[None, :]], axis=0)
    xbf_ref[...] = _pack_bf16(x)

    e0r = e0.reshape(NW, TPW)
    e1r = e1.reshape(NW, TPW)
    ce = lax.broadcasted_iota(jnp.int32, (NW, TPW, E), 2)
    h = (jnp.sum((e0r[:, :, None] == ce).astype(jnp.int32), axis=1)
         + jnp.sum((e1r[:, :, None] == ce).astype(jnp.int32), axis=1))
    hist_ref[...] = h

    totals = jnp.sum(h, axis=0)
    padded = (totals + (BLK - 1)) & ~(BLK - 1)
    ei = lax.broadcasted_iota(jnp.int32, (E, E), 0)
    ej = lax.broadcasted_iota(jnp.int32, (E, E), 1)
    start = jnp.sum(jnp.where(ej < ei, padded[None, :], 0), axis=1)

    bpos = lax.broadcasted_iota(jnp.int32, (NBLK, E), 0) * BLK
    ge = (bpos >= start[None, :]).astype(jnp.int32)
    emap = jnp.sum(ge, axis=1) - 1
    emap_ref[...] = emap

    bi = lax.broadcasted_iota(jnp.int32, (NBLK, NBLK), 0)
    bj = lax.broadcasted_iota(jnp.int32, (NBLK, NBLK), 1)
    prev = jnp.concatenate([emap[:1], emap[:-1]])
    chg = ((emap != prev)
           & (lax.broadcasted_iota(jnp.int32, (NBLK,), 0) > 0)).astype(jnp.int32)
    runx = jnp.sum(jnp.where(bj <= bi, chg[None, :], 0), axis=1)
    runx_ref[...] = runx
    nrun_ref[...] = runx[-1:] + 1

    rr = lax.broadcasted_iota(jnp.int32, (NBLK, E), 1)
    rexp_ref[...] = jnp.max(
        jnp.where(runx[:, None] == rr, emap[:, None], -1), axis=0)


def _gating(x, Wg):
    return pl.pallas_call(
        _gating_body,
        out_shape=[
            jax.ShapeDtypeStruct((T, K), jnp.float32),
            jax.ShapeDtypeStruct((K, T), jnp.int32),
            jax.ShapeDtypeStruct((NW, E), jnp.int32),
            jax.ShapeDtypeStruct((T, D2), jnp.int32),
            jax.ShapeDtypeStruct((NBLK,), jnp.int32),
            jax.ShapeDtypeStruct((NBLK,), jnp.int32),
            jax.ShapeDtypeStruct((E,), jnp.int32),
            jax.ShapeDtypeStruct((1,), jnp.int32),
        ],
    )(x, Wg)


def _route_body(e_hbm, hist_hbm, xbf_hbm, xs_hbm, pos_hbm,
                hist_v, e0_v, e1_v, pos0_v, pos1_v, xrows_v, sem):
    w = lax.axis_index("s") * 2 + lax.axis_index("c")
    lane = lax.broadcasted_iota(jnp.int32, (16,), 0)

    cps = [
        pltpu.async_copy(hist_hbm, hist_v, sem),
        pltpu.async_copy(e_hbm.at[0, pl.ds(w * TPW, TPW)], e0_v, sem),
        pltpu.async_copy(e_hbm.at[1, pl.ds(w * TPW, TPW)], e1_v, sem),
        pltpu.async_copy(xbf_hbm.at[pl.ds(w * TPW, TPW), :], xrows_v, sem),
    ]
    for cp in cps:
        cp.wait()

    totals = jnp.zeros((16,), jnp.int32)
    prefix = jnp.zeros((16,), jnp.int32)
    for c in range(NW):
        row = hist_v[c, :]
        totals = totals + row
        prefix = prefix + jnp.where(c < w, row, 0)

    padded = (totals + (BLK - 1)) & ~(BLK - 1)
    start = plsc.cumsum(padded) - padded
    counters = start + prefix

    for g in range(8):
        src = e0_v if g < 4 else e1_v
        ev = src[pl.ds((g % 4) * 16, 16)]
        posv = jnp.zeros((16,), jnp.int32)
        for e in range(E):
            msk = ev == e
            r = plsc.cumsum(jnp.where(msk, 1, 0))
            ce = jnp.sum(jnp.where(lane == e, counters, 0))
            posv = jnp.where(msk, ce + r - 1, posv)
            cnt = jnp.sum(jnp.where(msk, 1, 0))
            counters = jnp.where(lane == e, counters + cnt, counters)
        dst = pos0_v if g < 4 else pos1_v
        dst[pl.ds((g % 4) * 16, 16)] = posv

    cpos0 = pltpu.async_copy(pos0_v, pos_hbm.at[0, pl.ds(w * TPW, TPW)], sem)
    cpos1 = pltpu.async_copy(pos1_v, pos_hbm.at[1, pl.ds(w * TPW, TPW)], sem)

    c0 = pltpu.async_copy(xrows_v, xs_hbm.at[pos0_v], sem)
    c1 = pltpu.async_copy(xrows_v, xs_hbm.at[pos1_v], sem)
    cpos0.wait()
    cpos1.wait()
    c0.wait()
    c1.wait()


def _route(e_ids, hist, xbf):
    mesh = plsc.VectorSubcoreMesh(core_axis_name="c", subcore_axis_name="s")
    f = pl.kernel(
        _route_body,
        out_type=[
            jax.ShapeDtypeStruct((N_PAD, D2), jnp.int32),
            jax.ShapeDtypeStruct((K, T), jnp.int32),
        ],
        mesh=mesh,
        compiler_params=pltpu.CompilerParams(needs_layout_passes=False),
        scratch_types=[
            pltpu.VMEM((NW, E), jnp.int32),
            pltpu.VMEM((TPW,), jnp.int32),
            pltpu.VMEM((TPW,), jnp.int32),
            pltpu.VMEM((TPW,), jnp.int32),
            pltpu.VMEM((TPW,), jnp.int32),
            pltpu.VMEM((TPW, D2), jnp.int32),
            pltpu.SemaphoreType.DMA,
        ],
    )
    return f(e_ids, hist, xbf)


def _ffn_body(emap_ref, xs_ref, w1_ref, w2_ref, ys_ref):
    xb = _unpack_bf16_raw(xs_ref[...])
    h = jnp.maximum(
        jnp.dot(xb, w1_ref[0], preferred_element_type=jnp.float32), 0.0)
    y = jnp.dot(h.astype(jnp.bfloat16), w2_ref[0],
                preferred_element_type=jnp.float32)
    ys_ref[...] = _pack_bf16(y)


def _ffn(emap, xs, W1, W2):
    grid_spec = pltpu.PrefetchScalarGridSpec(
        num_scalar_prefetch=1,
        grid=(NBLK,),
        in_specs=[
            pl.BlockSpec((BLK, D2), lambda b, em: (b, 0)),
            pl.BlockSpec((1, D, D), lambda b, em: (em[b], 0, 0)),
            pl.BlockSpec((1, D, D), lambda b, em: (em[b], 0, 0)),
        ],
        out_specs=pl.BlockSpec((BLK, D2), lambda b, em: (b, 0)),
    )
    return pl.pallas_call(
        _ffn_body,
        grid_spec=grid_spec,
        out_shape=jax.ShapeDtypeStruct((N_PAD, D2), jnp.int32),
        compiler_params=pltpu.CompilerParams(
            dimension_semantics=("parallel",)),
    )(emap, xs, W1, W2)


def _gather_body(ys_hbm, pos_hbm, yg_hbm,
                 idx0_v, idx1_v, rows0_v, rows1_v, sem):
    w = lax.axis_index("s") * 2 + lax.axis_index("c")

    c0 = pltpu.async_copy(pos_hbm.at[0, pl.ds(w * TPW, TPW)], idx0_v, sem)
    c1 = pltpu.async_copy(pos_hbm.at[1, pl.ds(w * TPW, TPW)], idx1_v, sem)
    c0.wait()
    c1.wait()

    g0 = pltpu.async_copy(ys_hbm.at[idx0_v], rows0_v, sem)
    g1 = pltpu.async_copy(ys_hbm.at[idx1_v], rows1_v, sem)
    g0.wait()
    g1.wait()

    o0 = pltpu.async_copy(rows0_v, yg_hbm.at[0, pl.ds(w * TPW, TPW), :], sem)
    o1 = pltpu.async_copy(rows1_v, yg_hbm.at[1, pl.ds(w * TPW, TPW), :], sem)
    o0.wait()
    o1.wait()


def _gather(ys, pos):
    mesh = plsc.VectorSubcoreMesh(core_axis_name="c", subcore_axis_name="s")
    f = pl.kernel(
        _gather_body,
        out_type=jax.ShapeDtypeStruct((K, T, D2), jnp.int32),
        mesh=mesh,
        compiler_params=pltpu.CompilerParams(needs_layout_passes=False),
        scratch_types=[
            pltpu.VMEM((TPW,), jnp.int32),
            pltpu.VMEM((TPW,), jnp.int32),
            pltpu.VMEM((TPW, D2), jnp.int32),
            pltpu.VMEM((TPW, D2), jnp.int32),
            pltpu.SemaphoreType.DMA,
        ],
    )
    return f(ys, pos)


def _comb_body(p_ref, yg_ref, out_ref):
    y0 = _unpack_bf16(yg_ref[0])
    y1 = _unpack_bf16(yg_ref[1])
    out_ref[...] = p_ref[:, 0:1] * y0 + p_ref[:, 1:2] * y1


def _comb(p, yg):
    return pl.pallas_call(
        _comb_body,
        grid=(T // TCB,),
        in_specs=[
            pl.BlockSpec((TCB, K), lambda b: (b, 0)),
            pl.BlockSpec((K, TCB, D2), lambda b: (0, b, 0)),
        ],
        out_specs=pl.BlockSpec((TCB, D), lambda b: (b, 0)),
        out_shape=jax.ShapeDtypeStruct((T, D), jnp.float32),
        compiler_params=pltpu.CompilerParams(
            dimension_semantics=("parallel",)),
    )(p, yg)


@jax.jit
def kernel(x, Wg, W1, W2):
    p, e_ids, hist, xc, emap, _, _, _ = _gating(x, Wg)
    xs, pos = _route(e_ids, hist, xc)
    ys = _ffn(emap, xs, W1.astype(jnp.bfloat16), W2.astype(jnp.bfloat16))
    yg = _gather(ys, pos)
    return _comb(p, yg)


# speedup vs baseline: 1.1831x; 1.1831x over previous
import functools

import jax
import jax.numpy as jnp
from jax import lax
from jax.experimental import pallas as pl
from jax.experimental.pallas import tpu as pltpu
from jax.experimental.pallas import tpu_sc as plsc

T = 2048
D = 768
E = 16
K = 2
BLK = 128
NBLK = 48
N_PAD = NBLK * BLK
NW = 32
TPW = T // NW
APW = TPW * K
TCB = 512
D2 = D // 2


def _pack_bf16(a):
    ab = a.astype(jnp.bfloat16)
    lo = lax.bitcast_convert_type(ab[..., :D2], jnp.int16).astype(jnp.int32)
    hi = lax.bitcast_convert_type(ab[..., D2:], jnp.int16).astype(jnp.int32)
    return (hi << 16) | (lo & 0xFFFF)


def _unpack_bf16_raw(w):
    lo = lax.bitcast_convert_type(w.astype(jnp.int16), jnp.bfloat16)
    hi = lax.bitcast_convert_type((w >> 16).astype(jnp.int16), jnp.bfloat16)
    return jnp.concatenate([lo, hi], axis=-1)


def _unpack_bf16(w):
    return _unpack_bf16_raw(w).astype(jnp.float32)


def _gating_body(x_ref, wg_ref, p_ref, e_ref, hist_ref, xbf_ref,
                 emap_ref, runx_ref, rexp_ref, nrun_ref):
    x = x_ref[...]
    logits = jnp.dot(x, wg_ref[...], preferred_element_type=jnp.float32)
    m = jnp.max(logits, axis=-1, keepdims=True)
    ex = jnp.exp(logits - m)
    probs = ex / jnp.sum(ex, axis=-1, keepdims=True)

    iota = lax.broadcasted_iota(jnp.int32, (T, E), 1)
    p0 = jnp.max(probs, axis=-1)
    e0 = jnp.min(jnp.where(probs == p0[:, None], iota, E), axis=-1)
    probs2 = jnp.where(iota == e0[:, None], -jnp.inf, probs)
    p1 = jnp.max(probs2, axis=-1)
    e1 = jnp.min(jnp.where(probs2 == p1[:, None], iota, E), axis=-1)

    p_ref[...] = jnp.concatenate([p0[:, None], p1[:, None]], axis=1)
    e_ref[...] = jnp.concatenate([e0[None, :], e1[None, :]], axis=0)
    xbf_ref[...] = _pack_bf16(x)

    e0r = e0.reshape(NW, TPW)
    e1r = e1.reshape(NW, TPW)
    ce = lax.broadcasted_iota(jnp.int32, (NW, TPW, E), 2)
    h = (jnp.sum((e0r[:, :, None] == ce).astype(jnp.int32), axis=1)
         + jnp.sum((e1r[:, :, None] == ce).astype(jnp.int32), axis=1))
    hist_ref[...] = h

    totals = jnp.sum(h, axis=0)
    padded = (totals + (BLK - 1)) & ~(BLK - 1)
    ei = lax.broadcasted_iota(jnp.int32, (E, E), 0)
    ej = lax.broadcasted_iota(jnp.int32, (E, E), 1)
    start = jnp.sum(jnp.where(ej < ei, padded[None, :], 0), axis=1)

    bpos = lax.broadcasted_iota(jnp.int32, (NBLK, E), 0) * BLK
    ge = (bpos >= start[None, :]).astype(jnp.int32)
    emap = jnp.sum(ge, axis=1) - 1
    emap_ref[...] = emap

    bi = lax.broadcasted_iota(jnp.int32, (NBLK, NBLK), 0)
    bj = lax.broadcasted_iota(jnp.int32, (NBLK, NBLK), 1)
    prev = jnp.concatenate([emap[:1], emap[:-1]])
    chg = ((emap != prev)
           & (lax.broadcasted_iota(jnp.int32, (NBLK,), 0) > 0)).astype(jnp.int32)
    runx = jnp.sum(jnp.where(bj <= bi, chg[None, :], 0), axis=1)
    runx_ref[...] = runx
    nrun_ref[...] = runx[-1:] + 1

    rr = lax.broadcasted_iota(jnp.int32, (NBLK, E), 1)
    rexp_ref[...] = jnp.max(
        jnp.where(runx[:, None] == rr, emap[:, None], -1), axis=0)


def _gating(x, Wg):
    return pl.pallas_call(
        _gating_body,
        out_shape=[
            jax.ShapeDtypeStruct((T, K), jnp.float32),
            jax.ShapeDtypeStruct((K, T), jnp.int32),
            jax.ShapeDtypeStruct((NW, E), jnp.int32),
            jax.ShapeDtypeStruct((T, D2), jnp.int32),
            jax.ShapeDtypeStruct((NBLK,), jnp.int32),
            jax.ShapeDtypeStruct((NBLK,), jnp.int32),
            jax.ShapeDtypeStruct((E,), jnp.int32),
            jax.ShapeDtypeStruct((1,), jnp.int32),
        ],
    )(x, Wg)


def _route_body(e_hbm, hist_hbm, xbf_hbm, xs_hbm, pos_hbm,
                hist_v, e0_v, e1_v, pos0_v, pos1_v, xrows_v, sem):
    w = lax.axis_index("s") * 2 + lax.axis_index("c")
    lane = lax.broadcasted_iota(jnp.int32, (16,), 0)

    cps = [
        pltpu.async_copy(hist_hbm, hist_v, sem),
        pltpu.async_copy(e_hbm.at[0, pl.ds(w * TPW, TPW)], e0_v, sem),
        pltpu.async_copy(e_hbm.at[1, pl.ds(w * TPW, TPW)], e1_v, sem),
        pltpu.async_copy(xbf_hbm.at[pl.ds(w * TPW, TPW), :], xrows_v, sem),
    ]
    for cp in cps:
        cp.wait()

    totals = jnp.zeros((16,), jnp.int32)
    prefix = jnp.zeros((16,), jnp.int32)
    for c in range(NW):
        row = hist_v[c, :]
        totals = totals + row
        prefix = prefix + jnp.where(c < w, row, 0)

    padded = (totals + (BLK - 1)) & ~(BLK - 1)
    start = plsc.cumsum(padded) - padded
    counters = start + prefix

    for g in range(8):
        src = e0_v if g < 4 else e1_v
        ev = src[pl.ds((g % 4) * 16, 16)]
        posv = jnp.zeros((16,), jnp.int32)
        for e in range(E):
            msk = ev == e
            r = plsc.cumsum(jnp.where(msk, 1, 0))
            ce = jnp.sum(jnp.where(lane == e, counters, 0))
            posv = jnp.where(msk, ce + r - 1, posv)
            cnt = jnp.sum(jnp.where(msk, 1, 0))
            counters = jnp.where(lane == e, counters + cnt, counters)
        dst = pos0_v if g < 4 else pos1_v
        dst[pl.ds((g % 4) * 16, 16)] = posv

    cpos0 = pltpu.async_copy(pos0_v, pos_hbm.at[0, pl.ds(w * TPW, TPW)], sem)
    cpos1 = pltpu.async_copy(pos1_v, pos_hbm.at[1, pl.ds(w * TPW, TPW)], sem)

    c0 = pltpu.async_copy(xrows_v, xs_hbm.at[pos0_v], sem)
    c1 = pltpu.async_copy(xrows_v, xs_hbm.at[pos1_v], sem)
    cpos0.wait()
    cpos1.wait()
    c0.wait()
    c1.wait()


def _route(e_ids, hist, xbf):
    mesh = plsc.VectorSubcoreMesh(core_axis_name="c", subcore_axis_name="s")
    f = pl.kernel(
        _route_body,
        out_type=[
            jax.ShapeDtypeStruct((N_PAD, D2), jnp.int32),
            jax.ShapeDtypeStruct((K, T), jnp.int32),
        ],
        mesh=mesh,
        compiler_params=pltpu.CompilerParams(needs_layout_passes=False),
        scratch_types=[
            pltpu.VMEM((NW, E), jnp.int32),
            pltpu.VMEM((TPW,), jnp.int32),
            pltpu.VMEM((TPW,), jnp.int32),
            pltpu.VMEM((TPW,), jnp.int32),
            pltpu.VMEM((TPW,), jnp.int32),
            pltpu.VMEM((TPW, D2), jnp.int32),
            pltpu.SemaphoreType.DMA,
        ],
    )
    return f(e_ids, hist, xbf)


def _ffn_body(emap_ref, xs_ref, w1_ref, w2_ref, ys_ref):
    xb = _unpack_bf16_raw(xs_ref[...])
    h = jnp.maximum(
        jnp.dot(xb, w1_ref[0].astype(jnp.bfloat16),
                preferred_element_type=jnp.float32), 0.0)
    y = jnp.dot(h.astype(jnp.bfloat16), w2_ref[0].astype(jnp.bfloat16),
                preferred_element_type=jnp.float32)
    ys_ref[...] = _pack_bf16(y)


def _ffn(emap, xs, W1, W2):
    grid_spec = pltpu.PrefetchScalarGridSpec(
        num_scalar_prefetch=1,
        grid=(NBLK,),
        in_specs=[
            pl.BlockSpec((BLK, D2), lambda b, em: (b, 0)),
            pl.BlockSpec((1, D, D), lambda b, em: (em[b], 0, 0)),
            pl.BlockSpec((1, D, D), lambda b, em: (em[b], 0, 0)),
        ],
        out_specs=pl.BlockSpec((BLK, D2), lambda b, em: (b, 0)),
    )
    return pl.pallas_call(
        _ffn_body,
        grid_spec=grid_spec,
        out_shape=jax.ShapeDtypeStruct((N_PAD, D2), jnp.int32),
        compiler_params=pltpu.CompilerParams(
            dimension_semantics=("parallel",)),
    )(emap, xs, W1, W2)


def _gather_body(ys_hbm, pos_hbm, yg_hbm,
                 idx0_v, idx1_v, rows0_v, rows1_v, sem):
    w = lax.axis_index("s") * 2 + lax.axis_index("c")

    c0 = pltpu.async_copy(pos_hbm.at[0, pl.ds(w * TPW, TPW)], idx0_v, sem)
    c1 = pltpu.async_copy(pos_hbm.at[1, pl.ds(w * TPW, TPW)], idx1_v, sem)
    c0.wait()
    c1.wait()

    g0 = pltpu.async_copy(ys_hbm.at[idx0_v], rows0_v, sem)
    g1 = pltpu.async_copy(ys_hbm.at[idx1_v], rows1_v, sem)
    g0.wait()
    g1.wait()

    o0 = pltpu.async_copy(rows0_v, yg_hbm.at[0, pl.ds(w * TPW, TPW), :], sem)
    o1 = pltpu.async_copy(rows1_v, yg_hbm.at[1, pl.ds(w * TPW, TPW), :], sem)
    o0.wait()
    o1.wait()


def _gather(ys, pos):
    mesh = plsc.VectorSubcoreMesh(core_axis_name="c", subcore_axis_name="s")
    f = pl.kernel(
        _gather_body,
        out_type=jax.ShapeDtypeStruct((K, T, D2), jnp.int32),
        mesh=mesh,
        compiler_params=pltpu.CompilerParams(needs_layout_passes=False),
        scratch_types=[
            pltpu.VMEM((TPW,), jnp.int32),
            pltpu.VMEM((TPW,), jnp.int32),
            pltpu.VMEM((TPW, D2), jnp.int32),
            pltpu.VMEM((TPW, D2), jnp.int32),
            pltpu.SemaphoreType.DMA,
        ],
    )
    return f(ys, pos)


def _comb_body(p_ref, yg_ref, out_ref):
    y0 = _unpack_bf16(yg_ref[0])
    y1 = _unpack_bf16(yg_ref[1])
    out_ref[...] = p_ref[:, 0:1] * y0 + p_ref[:, 1:2] * y1


def _comb(p, yg):
    return pl.pallas_call(
        _comb_body,
        grid=(T // TCB,),
        in_specs=[
            pl.BlockSpec((TCB, K), lambda b: (b, 0)),
            pl.BlockSpec((K, TCB, D2), lambda b: (0, b, 0)),
        ],
        out_specs=pl.BlockSpec((TCB, D), lambda b: (b, 0)),
        out_shape=jax.ShapeDtypeStruct((T, D), jnp.float32),
        compiler_params=pltpu.CompilerParams(
            dimension_semantics=("parallel",)),
    )(p, yg)


@jax.jit
def kernel(x, Wg, W1, W2):
    p, e_ids, hist, xc, emap, _, _, _ = _gating(x, Wg)
    xs, pos = _route(e_ids, hist, xc)
    ys = _ffn(emap, xs, W1, W2)
    yg = _gather(ys, pos)
    return _comb(p, yg)


# speedup vs baseline: 1.1977x; 1.0124x over previous
import functools

import jax
import jax.numpy as jnp
from jax import lax
from jax.experimental import pallas as pl
from jax.experimental.pallas import tpu as pltpu
from jax.experimental.pallas import tpu_sc as plsc

T = 2048
D = 768
E = 16
K = 2
BLK = 128
NBLK = 48
N_PAD = NBLK * BLK
NW = 32
TPW = T // NW
APW = TPW * K
TCB = 512
D2 = D // 2


def _pack_bf16(a):
    ab = a.astype(jnp.bfloat16)
    lo = lax.bitcast_convert_type(ab[..., :D2], jnp.int16).astype(jnp.int32)
    hi = lax.bitcast_convert_type(ab[..., D2:], jnp.int16).astype(jnp.int32)
    return (hi << 16) | (lo & 0xFFFF)


def _unpack_bf16_raw(w):
    lo = lax.bitcast_convert_type(w.astype(jnp.int16), jnp.bfloat16)
    hi = lax.bitcast_convert_type((w >> 16).astype(jnp.int16), jnp.bfloat16)
    return jnp.concatenate([lo, hi], axis=-1)


def _unpack_bf16(w):
    return _unpack_bf16_raw(w).astype(jnp.float32)


def _gating_body(x_ref, wg_ref, p_ref, e_ref, hist_ref, xbf_ref,
                 emap_ref):
    x = x_ref[...]
    logits = jnp.dot(x, wg_ref[...], preferred_element_type=jnp.float32)
    m = jnp.max(logits, axis=-1, keepdims=True)
    ex = jnp.exp(logits - m)
    probs = ex / jnp.sum(ex, axis=-1, keepdims=True)

    iota = lax.broadcasted_iota(jnp.int32, (T, E), 1)
    p0 = jnp.max(probs, axis=-1)
    e0 = jnp.min(jnp.where(probs == p0[:, None], iota, E), axis=-1)
    probs2 = jnp.where(iota == e0[:, None], -jnp.inf, probs)
    p1 = jnp.max(probs2, axis=-1)
    e1 = jnp.min(jnp.where(probs2 == p1[:, None], iota, E), axis=-1)

    p_ref[...] = jnp.concatenate([p0[:, None], p1[:, None]], axis=1)
    e_ref[...] = jnp.concatenate([e0[None, :], e1[None, :]], axis=0)
    xbf_ref[...] = _pack_bf16(x)

    e0r = e0.reshape(NW, TPW)
    e1r = e1.reshape(NW, TPW)
    ce = lax.broadcasted_iota(jnp.int32, (NW, TPW, E), 2)
    h = (jnp.sum((e0r[:, :, None] == ce).astype(jnp.int32), axis=1)
         + jnp.sum((e1r[:, :, None] == ce).astype(jnp.int32), axis=1))
    hist_ref[...] = h

    totals = jnp.sum(h, axis=0)
    padded = (totals + (BLK - 1)) & ~(BLK - 1)
    ei = lax.broadcasted_iota(jnp.int32, (E, E), 0)
    ej = lax.broadcasted_iota(jnp.int32, (E, E), 1)
    start = jnp.sum(jnp.where(ej < ei, padded[None, :], 0), axis=1)

    bpos = lax.broadcasted_iota(jnp.int32, (NBLK, E), 0) * BLK
    ge = (bpos >= start[None, :]).astype(jnp.int32)
    emap_ref[...] = jnp.sum(ge, axis=1) - 1


def _gating(x, Wg):
    return pl.pallas_call(
        _gating_body,
        out_shape=[
            jax.ShapeDtypeStruct((T, K), jnp.float32),
            jax.ShapeDtypeStruct((K, T), jnp.int32),
            jax.ShapeDtypeStruct((NW, E), jnp.int32),
            jax.ShapeDtypeStruct((T, D2), jnp.int32),
            jax.ShapeDtypeStruct((NBLK,), jnp.int32),
        ],
    )(x, Wg)


def _route_body(e_hbm, hist_hbm, xbf_hbm, xs_hbm, pos_hbm,
                hist_v, e0_v, e1_v, pos0_v, pos1_v, xrows_v, sem):
    w = lax.axis_index("s") * 2 + lax.axis_index("c")
    lane = lax.broadcasted_iota(jnp.int32, (16,), 0)

    cps = [
        pltpu.async_copy(hist_hbm, hist_v, sem),
        pltpu.async_copy(e_hbm.at[0, pl.ds(w * TPW, TPW)], e0_v, sem),
        pltpu.async_copy(e_hbm.at[1, pl.ds(w * TPW, TPW)], e1_v, sem),
        pltpu.async_copy(xbf_hbm.at[pl.ds(w * TPW, TPW), :], xrows_v, sem),
    ]
    for cp in cps:
        cp.wait()

    totals = jnp.zeros((16,), jnp.int32)
    prefix = jnp.zeros((16,), jnp.int32)
    for c in range(NW):
        row = hist_v[c, :]
        totals = totals + row
        prefix = prefix + jnp.where(c < w, row, 0)

    padded = (totals + (BLK - 1)) & ~(BLK - 1)
    start = plsc.cumsum(padded) - padded
    counters = start + prefix

    for g in range(8):
        src = e0_v if g < 4 else e1_v
        ev = src[pl.ds((g % 4) * 16, 16)]
        posv = jnp.zeros((16,), jnp.int32)
        for e in range(E):
            msk = ev == e
            r = plsc.cumsum(jnp.where(msk, 1, 0))
            ce = jnp.sum(jnp.where(lane == e, counters, 0))
            posv = jnp.where(msk, ce + r - 1, posv)
            cnt = jnp.sum(jnp.where(msk, 1, 0))
            counters = jnp.where(lane == e, counters + cnt, counters)
        dst = pos0_v if g < 4 else pos1_v
        dst[pl.ds((g % 4) * 16, 16)] = posv

    cpos0 = pltpu.async_copy(pos0_v, pos_hbm.at[0, pl.ds(w * TPW, TPW)], sem)
    cpos1 = pltpu.async_copy(pos1_v, pos_hbm.at[1, pl.ds(w * TPW, TPW)], sem)

    c0 = pltpu.async_copy(xrows_v, xs_hbm.at[pos0_v], sem)
    c1 = pltpu.async_copy(xrows_v, xs_hbm.at[pos1_v], sem)
    cpos0.wait()
    cpos1.wait()
    c0.wait()
    c1.wait()


def _route(e_ids, hist, xbf):
    mesh = plsc.VectorSubcoreMesh(core_axis_name="c", subcore_axis_name="s")
    f = pl.kernel(
        _route_body,
        out_type=[
            jax.ShapeDtypeStruct((N_PAD, D2), jnp.int32),
            jax.ShapeDtypeStruct((K, T), jnp.int32),
        ],
        mesh=mesh,
        compiler_params=pltpu.CompilerParams(needs_layout_passes=False),
        scratch_types=[
            pltpu.VMEM((NW, E), jnp.int32),
            pltpu.VMEM((TPW,), jnp.int32),
            pltpu.VMEM((TPW,), jnp.int32),
            pltpu.VMEM((TPW,), jnp.int32),
            pltpu.VMEM((TPW,), jnp.int32),
            pltpu.VMEM((TPW, D2), jnp.int32),
            pltpu.SemaphoreType.DMA,
        ],
    )
    return f(e_ids, hist, xbf)


def _ffn_body(emap_ref, xs_ref, w1_ref, w2_ref, ys_ref):
    xb = _unpack_bf16(xs_ref[...])
    h = jnp.maximum(
        jnp.dot(xb, w1_ref[0], preferred_element_type=jnp.float32), 0.0)
    y = jnp.dot(h, w2_ref[0], preferred_element_type=jnp.float32)
    ys_ref[...] = _pack_bf16(y)


def _ffn(emap, xs, W1, W2):
    grid_spec = pltpu.PrefetchScalarGridSpec(
        num_scalar_prefetch=1,
        grid=(NBLK,),
        in_specs=[
            pl.BlockSpec((BLK, D2), lambda b, em: (b, 0)),
            pl.BlockSpec((1, D, D), lambda b, em: (em[b], 0, 0)),
            pl.BlockSpec((1, D, D), lambda b, em: (em[b], 0, 0)),
        ],
        out_specs=pl.BlockSpec((BLK, D2), lambda b, em: (b, 0)),
    )
    return pl.pallas_call(
        _ffn_body,
        grid_spec=grid_spec,
        out_shape=jax.ShapeDtypeStruct((N_PAD, D2), jnp.int32),
        compiler_params=pltpu.CompilerParams(
            dimension_semantics=("parallel",)),
    )(emap, xs, W1, W2)


def _gather_body(ys_hbm, pos_hbm, yg_hbm,
                 idx0_v, idx1_v, rows0_v, rows1_v, sem):
    w = lax.axis_index("s") * 2 + lax.axis_index("c")

    c0 = pltpu.async_copy(pos_hbm.at[0, pl.ds(w * TPW, TPW)], idx0_v, sem)
    c1 = pltpu.async_copy(pos_hbm.at[1, pl.ds(w * TPW, TPW)], idx1_v, sem)
    c0.wait()
    c1.wait()

    g0 = pltpu.async_copy(ys_hbm.at[idx0_v], rows0_v, sem)
    g1 = pltpu.async_copy(ys_hbm.at[idx1_v], rows1_v, sem)
    g0.wait()
    g1.wait()

    o0 = pltpu.async_copy(rows0_v, yg_hbm.at[0, pl.ds(w * TPW, TPW), :], sem)
    o1 = pltpu.async_copy(rows1_v, yg_hbm.at[1, pl.ds(w * TPW, TPW), :], sem)
    o0.wait()
    o1.wait()


def _gather(ys, pos):
    mesh = plsc.VectorSubcoreMesh(core_axis_name="c", subcore_axis_name="s")
    f = pl.kernel(
        _gather_body,
        out_type=jax.ShapeDtypeStruct((K, T, D2), jnp.int32),
        mesh=mesh,
        compiler_params=pltpu.CompilerParams(needs_layout_passes=False),
        scratch_types=[
            pltpu.VMEM((TPW,), jnp.int32),
            pltpu.VMEM((TPW,), jnp.int32),
            pltpu.VMEM((TPW, D2), jnp.int32),
            pltpu.VMEM((TPW, D2), jnp.int32),
            pltpu.SemaphoreType.DMA,
        ],
    )
    return f(ys, pos)


def _comb_body(p_ref, yg_ref, out_ref):
    y0 = _unpack_bf16(yg_ref[0])
    y1 = _unpack_bf16(yg_ref[1])
    out_ref[...] = p_ref[:, 0:1] * y0 + p_ref[:, 1:2] * y1


def _comb(p, yg):
    return pl.pallas_call(
        _comb_body,
        grid=(T // TCB,),
        in_specs=[
            pl.BlockSpec((TCB, K), lambda b: (b, 0)),
            pl.BlockSpec((K, TCB, D2), lambda b: (0, b, 0)),
        ],
        out_specs=pl.BlockSpec((TCB, D), lambda b: (b, 0)),
        out_shape=jax.ShapeDtypeStruct((T, D), jnp.float32),
        compiler_params=pltpu.CompilerParams(
            dimension_semantics=("parallel",)),
    )(p, yg)


@jax.jit
def kernel(x, Wg, W1, W2):
    p, e_ids, hist, xc, emap = _gating(x, Wg)
    xs, pos = _route(e_ids, hist, xc)
    ys = _ffn(emap, xs, W1, W2)
    yg = _gather(ys, pos)
    return _comb(p, yg)
